# baseline (speedup 1.0000x reference)
import functools

import jax
import jax.numpy as jnp
from jax import lax
from jax.experimental import pallas as pl
from jax.experimental.pallas import tpu as pltpu
from jax.experimental.pallas import tpu_sc as plsc

NN = 10000
CC = 128
EE = 320000
NC = 2
NS = 16
NW = NC * NS
CHUNK = 128
CPW = 80
IBLK = 8
NBLK = CPW // IBLK
K0 = 128
K1 = 32
E_PAD = NW * CPW * CHUNK
IDX_ROWS = E_PAD // CHUNK
N_ACC = 10112
STRIPE = N_ACC // NS

_MESH = dict(core_axis_name="c", subcore_axis_name="s", num_cores=NC,
             num_subcores=NS)


def _agg_body(x_hbm, srcm, dstm, zacc,
              acc_out,
              acc_sh, src_v, dst_v, rows_a, rows_b, sem_a, sem_b):
    c = lax.axis_index("c")
    s = lax.axis_index("s")
    pltpu.sync_copy(zacc.at[pl.ds(s * STRIPE, STRIPE)],
                    acc_sh.at[pl.ds(s * STRIPE, STRIPE)])
    plsc.subcore_barrier()

    base0 = s * (2 * CPW) + jnp.where(c == 0, 0, K0)
    nblk = jnp.where(c == 0, K0 // IBLK, K1 // IBLK)

    def ga(j):
        return pltpu.make_async_copy(x_hbm.at[src_v.at[j]], rows_a, sem_a)

    def gb(j):
        return pltpu.make_async_copy(x_hbm.at[src_v.at[j]], rows_b, sem_b)

    def outer(bi, carry):
        base = base0 + bi * IBLK
        pltpu.sync_copy(srcm.at[pl.ds(base, IBLK)], src_v)
        pltpu.sync_copy(dstm.at[pl.ds(base, IBLK)], dst_v)
        ga(0).start()

        def inner(u, c2):
            j = 2 * u
            gb(j + 1).start()
            ga(j).wait()
            pltpu.sync_copy(rows_a, acc_sh.at[dst_v.at[j]], add=True)
            ga(j + 2).start()
            gb(j + 1).wait()
            pltpu.sync_copy(rows_b, acc_sh.at[dst_v.at[j + 1]], add=True)
            return c2

        lax.fori_loop(0, IBLK // 2 - 1, inner, 0)
        j = IBLK - 2
        gb(j + 1).start()
        ga(j).wait()
        pltpu.sync_copy(rows_a, acc_sh.at[dst_v.at[j]], add=True)
        gb(j + 1).wait()
        pltpu.sync_copy(rows_b, acc_sh.at[dst_v.at[j + 1]], add=True)
        return carry

    lax.fori_loop(0, nblk, outer, 0)
    plsc.subcore_barrier()
    pltpu.sync_copy(acc_sh.at[pl.ds(s * STRIPE, STRIPE)],
                    acc_out.at[c, pl.ds(s * STRIPE, STRIPE)])


def _cnt_body(dstm, zcnt, ones_hbm,
              cnt_out,
              cnt_sh, dst_v, ones_v):
    c = lax.axis_index("c")
    s = lax.axis_index("s")
    w = s * NC + c
    pltpu.sync_copy(zcnt.at[pl.ds(s * STRIPE, STRIPE)],
                    cnt_sh.at[pl.ds(s * STRIPE, STRIPE)])
    pltpu.sync_copy(ones_hbm, ones_v)
    plsc.subcore_barrier()

    def outer(bi, carry):
        base = w * CPW + bi * IBLK
        pltpu.sync_copy(dstm.at[pl.ds(base, IBLK)], dst_v)

        def inner(j, c2):
            pltpu.sync_copy(ones_v, cnt_sh.at[dst_v.at[j]], add=True)
            return c2

        lax.fori_loop(0, IBLK, inner, 0)
        return carry

    lax.fori_loop(0, NBLK, outer, 0)
    plsc.subcore_barrier()
    pltpu.sync_copy(cnt_sh.at[pl.ds(s * STRIPE, STRIPE)],
                    cnt_out.at[c, pl.ds(s * STRIPE, STRIPE)])


_sc_agg = pl.kernel(
    _agg_body,
    out_type=jax.ShapeDtypeStruct((NC, N_ACC, CC), jnp.float32),
    mesh=plsc.VectorSubcoreMesh(**_MESH),
    scratch_types=[
        pltpu.VMEM_SHARED((N_ACC, CC), jnp.float32),
        pltpu.VMEM((IBLK, CHUNK), jnp.int32),
        pltpu.VMEM((IBLK, CHUNK), jnp.int32),
        pltpu.VMEM((CHUNK, CC), jnp.float32),
        pltpu.VMEM((CHUNK, CC), jnp.float32),
        pltpu.SemaphoreType.DMA,
        pltpu.SemaphoreType.DMA,
    ],
)

_sc_cnt = pl.kernel(
    _cnt_body,
    out_type=jax.ShapeDtypeStruct((NC, N_ACC, CC), jnp.float32),
    mesh=plsc.VectorSubcoreMesh(**_MESH),
    scratch_types=[
        pltpu.VMEM_SHARED((N_ACC, CC), jnp.float32),
        pltpu.VMEM((IBLK, CHUNK), jnp.int32),
        pltpu.VMEM((CHUNK, CC), jnp.float32),
    ],
)


def _tc_layer_body(relu, acc_ref, cnt_ref, x_ref, wl_ref, wr_ref, b_ref, o_ref):
    acc = acc_ref[0] + acc_ref[1]
    cnt = cnt_ref[0] + cnt_ref[1]
    denom = jnp.maximum(cnt[:, 0:1], 1.0)
    agg = acc / denom
    h = (jnp.dot(agg, wl_ref[...], preferred_element_type=jnp.float32)
         + jnp.dot(x_ref[...], wr_ref[...], preferred_element_type=jnp.float32)
         + b_ref[...])
    o_ref[...] = jnp.maximum(h, 0.0) if relu else h


def _tc_layer(relu, acc, cnt, x, wl, wr, b):
    blk = 1000
    grid = (NN // blk,)
    return pl.pallas_call(
        functools.partial(_tc_layer_body, relu),
        grid=grid,
        in_specs=[
            pl.BlockSpec((NC, blk, CC), lambda i: (0, i, 0)),
            pl.BlockSpec((NC, blk, CC), lambda i: (0, i, 0)),
            pl.BlockSpec((blk, CC), lambda i: (i, 0)),
            pl.BlockSpec((CC, CC), lambda i: (0, 0)),
            pl.BlockSpec((CC, CC), lambda i: (0, 0)),
            pl.BlockSpec((1, CC), lambda i: (0, 0)),
        ],
        out_specs=pl.BlockSpec((blk, CC), lambda i: (i, 0)),
        out_shape=jax.ShapeDtypeStruct((NN, CC), jnp.float32),
    )(acc, cnt, x, wl, wr, b)


def kernel(x, edge_index, W1_l, W1_r, b1, W2_l, W2_r, b2):
    src = edge_index[0]
    dst = edge_index[1]
    pad = E_PAD - EE
    srcm = jnp.concatenate([src, jnp.zeros((pad,), jnp.int32)]).reshape(
        IDX_ROWS, CHUNK)
    dstm = jnp.concatenate([dst, jnp.full((pad,), NN, jnp.int32)]).reshape(
        IDX_ROWS, CHUNK)
    zacc = jnp.zeros((N_ACC, CC), jnp.float32)
    ones = jnp.ones((CHUNK, CC), jnp.float32)

    cnt = _sc_cnt(dstm, zacc, ones)
    acc1 = _sc_agg(x, srcm, dstm, zacc)
    h = _tc_layer(True, acc1, cnt, x, W1_l, W1_r, b1.reshape(1, CC))
    acc2 = _sc_agg(h, srcm, dstm, zacc)
    out = _tc_layer(False, acc2, cnt, h, W2_l, W2_r, b2.reshape(1, CC))
    return out


# speedup vs baseline: 1.1641x; 1.1641x over previous
import functools

import jax
import jax.numpy as jnp
from jax import lax
from jax.experimental import pallas as pl
from jax.experimental.pallas import tpu as pltpu
from jax.experimental.pallas import tpu_sc as plsc

NN = 10000
CC = 128
EE = 320000
NC = 2
NS = 16
NW = NC * NS
CHUNK = 128
CPW = 80
IBLK = 8
NBLK = CPW // IBLK
K0 = 120
K1 = 40
E_PAD = NW * CPW * CHUNK
IDX_ROWS = E_PAD // CHUNK
N_ACC = 10112
STRIPE = N_ACC // NS

_MESH = dict(core_axis_name="c", subcore_axis_name="s", num_cores=NC,
             num_subcores=NS)


def _agg_body(x_hbm, x2_hbm, srcm, dstm, zacc,
              acc_out,
              acc_sh, src_v, dst_v, rows_a, rows_b, sem_a, sem_b):
    c = lax.axis_index("c")
    s = lax.axis_index("s")
    pltpu.sync_copy(zacc.at[pl.ds(s * STRIPE, STRIPE)],
                    acc_sh.at[pl.ds(s * STRIPE, STRIPE)])
    plsc.subcore_barrier()

    base0 = s * (2 * CPW) + jnp.where(c == 0, 0, K0)
    nblk = jnp.where(c == 0, K0 // IBLK, K1 // IBLK)

    def run(table, my_nblk):
        def ga(j):
            return pltpu.make_async_copy(table.at[src_v.at[j]], rows_a, sem_a)

        def gb(j):
            return pltpu.make_async_copy(table.at[src_v.at[j]], rows_b, sem_b)

        def outer(bi, carry):
            base = base0 + bi * IBLK
            pltpu.sync_copy(srcm.at[pl.ds(base, IBLK)], src_v)
            pltpu.sync_copy(dstm.at[pl.ds(base, IBLK)], dst_v)
            ga(0).start()

            def inner(u, c2):
                j = 2 * u
                gb(j + 1).start()
                ga(j).wait()
                pltpu.sync_copy(rows_a, acc_sh.at[dst_v.at[j]], add=True)
                ga(j + 2).start()
                gb(j + 1).wait()
                pltpu.sync_copy(rows_b, acc_sh.at[dst_v.at[j + 1]], add=True)
                return c2

            lax.fori_loop(0, IBLK // 2 - 1, inner, 0)
            j = IBLK - 2
            gb(j + 1).start()
            ga(j).wait()
            pltpu.sync_copy(rows_a, acc_sh.at[dst_v.at[j]], add=True)
            gb(j + 1).wait()
            pltpu.sync_copy(rows_b, acc_sh.at[dst_v.at[j + 1]], add=True)
            return carry

        lax.fori_loop(0, my_nblk, outer, 0)

    @pl.when(c == 0)
    def _():
        run(x_hbm, K0 // IBLK)

    @pl.when(c != 0)
    def _():
        run(x2_hbm, K1 // IBLK)

    plsc.subcore_barrier()
    pltpu.sync_copy(acc_sh.at[pl.ds(s * STRIPE, STRIPE)],
                    acc_out.at[c, pl.ds(s * STRIPE, STRIPE)])


def _cnt_body(dstm, zcnt, ones_hbm,
              cnt_out,
              cnt_sh, dst_v, ones_v):
    c = lax.axis_index("c")
    s = lax.axis_index("s")
    w = s * NC + c
    pltpu.sync_copy(zcnt.at[pl.ds(s * STRIPE, STRIPE)],
                    cnt_sh.at[pl.ds(s * STRIPE, STRIPE)])
    pltpu.sync_copy(ones_hbm, ones_v)
    plsc.subcore_barrier()

    def outer(bi, carry):
        base = w * CPW + bi * IBLK
        pltpu.sync_copy(dstm.at[pl.ds(base, IBLK)], dst_v)

        def inner(j, c2):
            pltpu.sync_copy(ones_v, cnt_sh.at[dst_v.at[j]], add=True)
            return c2

        lax.fori_loop(0, IBLK, inner, 0)
        return carry

    lax.fori_loop(0, NBLK, outer, 0)
    plsc.subcore_barrier()
    pltpu.sync_copy(cnt_sh.at[pl.ds(s * STRIPE, STRIPE)],
                    cnt_out.at[c, pl.ds(s * STRIPE, STRIPE)])


_sc_agg = pl.kernel(
    _agg_body,
    out_type=jax.ShapeDtypeStruct((NC, N_ACC, CC), jnp.float32),
    mesh=plsc.VectorSubcoreMesh(**_MESH),
    scratch_types=[
        pltpu.VMEM_SHARED((N_ACC, CC), jnp.float32),
        pltpu.VMEM((IBLK, CHUNK), jnp.int32),
        pltpu.VMEM((IBLK, CHUNK), jnp.int32),
        pltpu.VMEM((CHUNK, CC), jnp.float32),
        pltpu.VMEM((CHUNK, CC), jnp.float32),
        pltpu.SemaphoreType.DMA,
        pltpu.SemaphoreType.DMA,
    ],
)

_sc_cnt = pl.kernel(
    _cnt_body,
    out_type=jax.ShapeDtypeStruct((NC, N_ACC, CC), jnp.float32),
    mesh=plsc.VectorSubcoreMesh(**_MESH),
    scratch_types=[
        pltpu.VMEM_SHARED((N_ACC, CC), jnp.float32),
        pltpu.VMEM((IBLK, CHUNK), jnp.int32),
        pltpu.VMEM((CHUNK, CC), jnp.float32),
    ],
)


def _tc_layer_body(relu, acc_ref, cnt_ref, x_ref, wl_ref, wr_ref, b_ref, o_ref):
    acc = acc_ref[0] + acc_ref[1]
    cnt = cnt_ref[0] + cnt_ref[1]
    denom = jnp.maximum(cnt[:, 0:1], 1.0)
    agg = acc / denom
    h = (jnp.dot(agg, wl_ref[...], preferred_element_type=jnp.float32)
         + jnp.dot(x_ref[...], wr_ref[...], preferred_element_type=jnp.float32)
         + b_ref[...])
    o_ref[...] = jnp.maximum(h, 0.0) if relu else h


def _tc_layer(relu, acc, cnt, x, wl, wr, b):
    blk = 1000
    grid = (NN // blk,)
    return pl.pallas_call(
        functools.partial(_tc_layer_body, relu),
        grid=grid,
        in_specs=[
            pl.BlockSpec((NC, blk, CC), lambda i: (0, i, 0)),
            pl.BlockSpec((NC, blk, CC), lambda i: (0, i, 0)),
            pl.BlockSpec((blk, CC), lambda i: (i, 0)),
            pl.BlockSpec((CC, CC), lambda i: (0, 0)),
            pl.BlockSpec((CC, CC), lambda i: (0, 0)),
            pl.BlockSpec((1, CC), lambda i: (0, 0)),
        ],
        out_specs=pl.BlockSpec((blk, CC), lambda i: (i, 0)),
        out_shape=jax.ShapeDtypeStruct((NN, CC), jnp.float32),
    )(acc, cnt, x, wl, wr, b)


def kernel(x, edge_index, W1_l, W1_r, b1, W2_l, W2_r, b2):
    src = edge_index[0]
    dst = edge_index[1]
    pad = E_PAD - EE
    srcm = jnp.concatenate([src, jnp.zeros((pad,), jnp.int32)]).reshape(
        IDX_ROWS, CHUNK)
    dstm = jnp.concatenate([dst, jnp.full((pad,), NN, jnp.int32)]).reshape(
        IDX_ROWS, CHUNK)
    zacc = jnp.zeros((N_ACC, CC), jnp.float32)
    ones = jnp.ones((CHUNK, CC), jnp.float32)

    z0 = lax.optimization_barrier(jnp.float32(0.0))
    x2 = x + z0
    cnt = _sc_cnt(dstm, zacc, ones)
    acc1 = _sc_agg(x, x2, srcm, dstm, zacc)
    h = _tc_layer(True, acc1, cnt, x, W1_l, W1_r, b1.reshape(1, CC))
    h2 = h + z0
    acc2 = _sc_agg(h, h2, srcm, dstm, zacc)
    out = _tc_layer(False, acc2, cnt, h, W2_l, W2_r, b2.reshape(1, CC))
    return out


# speedup vs baseline: 1.2949x; 1.1124x over previous
import functools

import jax
import jax.numpy as jnp
from jax import lax
from jax.experimental import pallas as pl
from jax.experimental.pallas import tpu as pltpu
from jax.experimental.pallas import tpu_sc as plsc

NN = 10000
CC = 128
EE = 320000
NC = 2
NS = 16
NW = NC * NS
CHUNK = 128
CPW = 80
IBLK = 8
NBLK = CPW // IBLK
K0 = 120
K1 = 40
E_PAD = NW * CPW * CHUNK
IDX_ROWS = E_PAD // CHUNK
N_ACC = 10112
STRIPE = N_ACC // NS

_MESH = dict(core_axis_name="c", subcore_axis_name="s", num_cores=NC,
             num_subcores=NS)


def _agg_body(x_hbm, x2_hbm, x3_hbm, x4_hbm, srcm, dstm, zacc,
              acc_out,
              acc_sh, src_v, dst_v, rows_a, rows_b, sem_a, sem_b):
    c = lax.axis_index("c")
    s = lax.axis_index("s")
    pltpu.sync_copy(zacc.at[pl.ds(s * STRIPE, STRIPE)],
                    acc_sh.at[pl.ds(s * STRIPE, STRIPE)])
    plsc.subcore_barrier()

    base0 = s * (2 * CPW) + jnp.where(c == 0, 0, K0)
    nblk = jnp.where(c == 0, K0 // IBLK, K1 // IBLK)

    def run(table, table2, my_nblk):
        def ga(j):
            return pltpu.make_async_copy(table.at[src_v.at[j]], rows_a, sem_a)

        def gb(j):
            return pltpu.make_async_copy(table2.at[src_v.at[j]], rows_b, sem_b)

        def outer(bi, carry):
            base = base0 + bi * IBLK
            pltpu.sync_copy(srcm.at[pl.ds(base, IBLK)], src_v)
            pltpu.sync_copy(dstm.at[pl.ds(base, IBLK)], dst_v)
            ga(0).start()

            def inner(u, c2):
                j = 2 * u
                gb(j + 1).start()
                ga(j).wait()
                pltpu.sync_copy(rows_a, acc_sh.at[dst_v.at[j]], add=True)
                ga(j + 2).start()
                gb(j + 1).wait()
                pltpu.sync_copy(rows_b, acc_sh.at[dst_v.at[j + 1]], add=True)
                return c2

            lax.fori_loop(0, IBLK // 2 - 1, inner, 0)
            j = IBLK - 2
            gb(j + 1).start()
            ga(j).wait()
            pltpu.sync_copy(rows_a, acc_sh.at[dst_v.at[j]], add=True)
            gb(j + 1).wait()
            pltpu.sync_copy(rows_b, acc_sh.at[dst_v.at[j + 1]], add=True)
            return carry

        lax.fori_loop(0, my_nblk, outer, 0)

    @pl.when(c == 0)
    def _():
        run(x_hbm, x3_hbm, K0 // IBLK)

    @pl.when(c != 0)
    def _():
        run(x2_hbm, x4_hbm, K1 // IBLK)

    plsc.subcore_barrier()
    pltpu.sync_copy(acc_sh.at[pl.ds(s * STRIPE, STRIPE)],
                    acc_out.at[c, pl.ds(s * STRIPE, STRIPE)])


def _cnt_body(dstm, zcnt, ones_hbm,
              cnt_out,
              cnt_sh, dst_v, ones_v):
    c = lax.axis_index("c")
    s = lax.axis_index("s")
    w = s * NC + c
    pltpu.sync_copy(zcnt.at[pl.ds(s * STRIPE, STRIPE)],
                    cnt_sh.at[pl.ds(s * STRIPE, STRIPE)])
    pltpu.sync_copy(ones_hbm, ones_v)
    plsc.subcore_barrier()

    def outer(bi, carry):
        base = w * CPW + bi * IBLK
        pltpu.sync_copy(dstm.at[pl.ds(base, IBLK)], dst_v)

        def inner(j, c2):
            pltpu.sync_copy(ones_v, cnt_sh.at[dst_v.at[j]], add=True)
            return c2

        lax.fori_loop(0, IBLK, inner, 0)
        return carry

    lax.fori_loop(0, NBLK, outer, 0)
    plsc.subcore_barrier()
    pltpu.sync_copy(cnt_sh.at[pl.ds(s * STRIPE, STRIPE)],
                    cnt_out.at[c, pl.ds(s * STRIPE, STRIPE)])


_sc_agg = pl.kernel(
    _agg_body,
    out_type=jax.ShapeDtypeStruct((NC, N_ACC, CC), jnp.float32),
    mesh=plsc.VectorSubcoreMesh(**_MESH),
    scratch_types=[
        pltpu.VMEM_SHARED((N_ACC, CC), jnp.float32),
        pltpu.VMEM((IBLK, CHUNK), jnp.int32),
        pltpu.VMEM((IBLK, CHUNK), jnp.int32),
        pltpu.VMEM((CHUNK, CC), jnp.float32),
        pltpu.VMEM((CHUNK, CC), jnp.float32),
        pltpu.SemaphoreType.DMA,
        pltpu.SemaphoreType.DMA,
    ],
)

_sc_cnt = pl.kernel(
    _cnt_body,
    out_type=jax.ShapeDtypeStruct((NC, N_ACC, CC), jnp.float32),
    mesh=plsc.VectorSubcoreMesh(**_MESH),
    scratch_types=[
        pltpu.VMEM_SHARED((N_ACC, CC), jnp.float32),
        pltpu.VMEM((IBLK, CHUNK), jnp.int32),
        pltpu.VMEM((CHUNK, CC), jnp.float32),
    ],
)


def _tc_layer_body(relu, acc_ref, cnt_ref, x_ref, wl_ref, wr_ref, b_ref, o_ref):
    acc = acc_ref[0] + acc_ref[1]
    cnt = cnt_ref[0] + cnt_ref[1]
    denom = jnp.maximum(cnt[:, 0:1], 1.0)
    agg = acc / denom
    h = (jnp.dot(agg, wl_ref[...], preferred_element_type=jnp.float32)
         + jnp.dot(x_ref[...], wr_ref[...], preferred_element_type=jnp.float32)
         + b_ref[...])
    o_ref[...] = jnp.maximum(h, 0.0) if relu else h


def _tc_layer(relu, acc, cnt, x, wl, wr, b):
    blk = 1000
    grid = (NN // blk,)
    return pl.pallas_call(
        functools.partial(_tc_layer_body, relu),
        grid=grid,
        in_specs=[
            pl.BlockSpec((NC, blk, CC), lambda i: (0, i, 0)),
            pl.BlockSpec((NC, blk, CC), lambda i: (0, i, 0)),
            pl.BlockSpec((blk, CC), lambda i: (i, 0)),
            pl.BlockSpec((CC, CC), lambda i: (0, 0)),
            pl.BlockSpec((CC, CC), lambda i: (0, 0)),
            pl.BlockSpec((1, CC), lambda i: (0, 0)),
        ],
        out_specs=pl.BlockSpec((blk, CC), lambda i: (i, 0)),
        out_shape=jax.ShapeDtypeStruct((NN, CC), jnp.float32),
    )(acc, cnt, x, wl, wr, b)


def kernel(x, edge_index, W1_l, W1_r, b1, W2_l, W2_r, b2):
    src = edge_index[0]
    dst = edge_index[1]
    pad = E_PAD - EE
    srcm = jnp.concatenate([src, jnp.zeros((pad,), jnp.int32)]).reshape(
        IDX_ROWS, CHUNK)
    dstm = jnp.concatenate([dst, jnp.full((pad,), NN, jnp.int32)]).reshape(
        IDX_ROWS, CHUNK)
    zacc = jnp.zeros((N_ACC, CC), jnp.float32)
    ones = jnp.ones((CHUNK, CC), jnp.float32)

    z0 = lax.optimization_barrier(jnp.float32(0.0))
    zs = lax.optimization_barrier((jnp.float32(0.0), jnp.float32(0.0),
                                   jnp.float32(0.0)))
    x2, x3, x4 = x + zs[0], x + zs[1], x + zs[2]
    cnt = _sc_cnt(dstm, zacc, ones)
    acc1 = _sc_agg(x, x2, x3, x4, srcm, dstm, zacc)
    h = _tc_layer(True, acc1, cnt, x, W1_l, W1_r, b1.reshape(1, CC))
    h2, h3, h4 = h + zs[0], h + zs[1], h + zs[2]
    acc2 = _sc_agg(h, h2, h3, h4, srcm, dstm, zacc)
    out = _tc_layer(False, acc2, cnt, h, W2_l, W2_r, b2.reshape(1, CC))
    return out


# speedup vs baseline: 1.3207x; 1.0199x over previous
import functools

import jax
import jax.numpy as jnp
from jax import lax
from jax.experimental import pallas as pl
from jax.experimental.pallas import tpu as pltpu
from jax.experimental.pallas import tpu_sc as plsc

NN = 10000
CC = 128
EE = 320000
NC = 2
NS = 16
NW = NC * NS
CHUNK = 80
KT = 256
K0 = 192
K1 = 64
IBLK = 8
NRING = 3
E_PAD = NS * KT * CHUNK
IDX_ROWS = E_PAD // CHUNK
CPWC = IDX_ROWS // NW
N_ACC = 10112
STRIPE = N_ACC // NS

_MESH = dict(core_axis_name="c", subcore_axis_name="s", num_cores=NC,
             num_subcores=NS)


def _agg_body(t1, t2, t3, t4, t5, t6, srcm, dstm, zacc,
              acc_out,
              acc_sh, src_v, dst_v, r0, r1, r2, s0, s1, s2):
    c = lax.axis_index("c")
    s = lax.axis_index("s")
    pltpu.sync_copy(zacc.at[pl.ds(s * STRIPE, STRIPE)],
                    acc_sh.at[pl.ds(s * STRIPE, STRIPE)])
    plsc.subcore_barrier()

    base0 = s * KT + jnp.where(c == 0, 0, K0)
    rows = (r0, r1, r2)
    sems = (s0, s1, s2)

    def run(tabs, my_nblk):
        def g(b, j):
            return pltpu.make_async_copy(tabs[b].at[src_v.at[j]], rows[b],
                                         sems[b])

        def outer(bi, carry):
            base = base0 + bi * IBLK
            pltpu.sync_copy(srcm.at[pl.ds(base, IBLK)], src_v)
            pltpu.sync_copy(dstm.at[pl.ds(base, IBLK)], dst_v)
            for j in range(NRING):
                g(j, j).start()
            for j in range(IBLK):
                b = j % NRING
                g(b, j).wait()
                pltpu.sync_copy(rows[b], acc_sh.at[dst_v.at[j]], add=True)
                if j + NRING < IBLK:
                    g(b, j + NRING).start()
            return carry

        lax.fori_loop(0, my_nblk, outer, 0)

    @pl.when(c == 0)
    def _():
        run((t1, t3, t5), K0 // IBLK)

    @pl.when(c != 0)
    def _():
        run((t2, t4, t6), K1 // IBLK)

    plsc.subcore_barrier()
    pltpu.sync_copy(acc_sh.at[pl.ds(s * STRIPE, STRIPE)],
                    acc_out.at[c, pl.ds(s * STRIPE, STRIPE)])


def _cnt_body(dstm, zcnt, ones_hbm,
              cnt_out,
              cnt_sh, dst_v, ones_v):
    c = lax.axis_index("c")
    s = lax.axis_index("s")
    w = s * NC + c
    pltpu.sync_copy(zcnt.at[pl.ds(s * STRIPE, STRIPE)],
                    cnt_sh.at[pl.ds(s * STRIPE, STRIPE)])
    pltpu.sync_copy(ones_hbm, ones_v)
    plsc.subcore_barrier()

    def outer(bi, carry):
        base = w * CPWC + bi * IBLK
        pltpu.sync_copy(dstm.at[pl.ds(base, IBLK)], dst_v)

        def inner(j, c2):
            pltpu.sync_copy(ones_v, cnt_sh.at[dst_v.at[j]], add=True)
            return c2

        lax.fori_loop(0, IBLK, inner, 0)
        return carry

    lax.fori_loop(0, CPWC // IBLK, outer, 0)
    plsc.subcore_barrier()
    pltpu.sync_copy(cnt_sh.at[pl.ds(s * STRIPE, STRIPE)],
                    cnt_out.at[c, pl.ds(s * STRIPE, STRIPE)])


_sc_agg = pl.kernel(
    _agg_body,
    out_type=jax.ShapeDtypeStruct((NC, N_ACC, CC), jnp.float32),
    mesh=plsc.VectorSubcoreMesh(**_MESH),
    scratch_types=[
        pltpu.VMEM_SHARED((N_ACC, CC), jnp.float32),
        pltpu.VMEM((IBLK, CHUNK), jnp.int32),
        pltpu.VMEM((IBLK, CHUNK), jnp.int32),
        pltpu.VMEM((CHUNK, CC), jnp.float32),
        pltpu.VMEM((CHUNK, CC), jnp.float32),
        pltpu.VMEM((CHUNK, CC), jnp.float32),
        pltpu.SemaphoreType.DMA,
        pltpu.SemaphoreType.DMA,
        pltpu.SemaphoreType.DMA,
    ],
)

_sc_cnt = pl.kernel(
    _cnt_body,
    out_type=jax.ShapeDtypeStruct((NC, N_ACC, CC), jnp.float32),
    mesh=plsc.VectorSubcoreMesh(**_MESH),
    scratch_types=[
        pltpu.VMEM_SHARED((N_ACC, CC), jnp.float32),
        pltpu.VMEM((IBLK, CHUNK), jnp.int32),
        pltpu.VMEM((CHUNK, CC), jnp.float32),
    ],
)


def _tc_layer_body(relu, acc_ref, cnt_ref, x_ref, wl_ref, wr_ref, b_ref, o_ref):
    acc = acc_ref[0] + acc_ref[1]
    cnt = cnt_ref[0] + cnt_ref[1]
    denom = jnp.maximum(cnt[:, 0:1], 1.0)
    agg = acc / denom
    h = (jnp.dot(agg, wl_ref[...], preferred_element_type=jnp.float32)
         + jnp.dot(x_ref[...], wr_ref[...], preferred_element_type=jnp.float32)
         + b_ref[...])
    o_ref[...] = jnp.maximum(h, 0.0) if relu else h


def _tc_layer(relu, acc, cnt, x, wl, wr, b):
    blk = 1000
    grid = (NN // blk,)
    return pl.pallas_call(
        functools.partial(_tc_layer_body, relu),
        grid=grid,
        in_specs=[
            pl.BlockSpec((NC, blk, CC), lambda i: (0, i, 0)),
            pl.BlockSpec((NC, blk, CC), lambda i: (0, i, 0)),
            pl.BlockSpec((blk, CC), lambda i: (i, 0)),
            pl.BlockSpec((CC, CC), lambda i: (0, 0)),
            pl.BlockSpec((CC, CC), lambda i: (0, 0)),
            pl.BlockSpec((1, CC), lambda i: (0, 0)),
        ],
        out_specs=pl.BlockSpec((blk, CC), lambda i: (i, 0)),
        out_shape=jax.ShapeDtypeStruct((NN, CC), jnp.float32),
    )(acc, cnt, x, wl, wr, b)


def kernel(x, edge_index, W1_l, W1_r, b1, W2_l, W2_r, b2):
    src = edge_index[0]
    dst = edge_index[1]
    pad = E_PAD - EE
    srcm = jnp.concatenate([src, jnp.zeros((pad,), jnp.int32)]).reshape(
        IDX_ROWS, CHUNK)
    dstm = jnp.concatenate([dst, jnp.full((pad,), NN, jnp.int32)]).reshape(
        IDX_ROWS, CHUNK)
    zacc = jnp.zeros((N_ACC, CC), jnp.float32)
    ones = jnp.ones((CHUNK, CC), jnp.float32)

    zs = lax.optimization_barrier(tuple(jnp.float32(0.0) for _ in range(5)))
    xs = tuple(x + z for z in zs)
    cnt = _sc_cnt(dstm, zacc, ones)
    acc1 = _sc_agg(x, *xs, srcm, dstm, zacc)
    h = _tc_layer(True, acc1, cnt, x, W1_l, W1_r, b1.reshape(1, CC))
    hs = tuple(h + z for z in zs)
    acc2 = _sc_agg(h, *hs, srcm, dstm, zacc)
    out = _tc_layer(False, acc2, cnt, h, W2_l, W2_r, b2.reshape(1, CC))
    return out


# speedup vs baseline: 1.5591x; 1.1805x over previous
import functools

import jax
import jax.numpy as jnp
from jax import lax
from jax.experimental import pallas as pl
from jax.experimental.pallas import tpu as pltpu
from jax.experimental.pallas import tpu_sc as plsc

NN = 10000
CC = 128
EE = 320000
NC = 2
NS = 16
NW = NC * NS
CHUNK = 80
KT = 256
K0 = 176
K1 = 80
IBLK = 8
NRING = 3
E_PAD = NS * KT * CHUNK
IDX_ROWS = E_PAD // CHUNK
CHUNKC = 128
IDX_ROWS_C = E_PAD // CHUNKC
CPWC = IDX_ROWS_C // NW
N_ACC = 10112
STRIPE = N_ACC // NS

_MESH = dict(core_axis_name="c", subcore_axis_name="s", num_cores=NC,
             num_subcores=NS)


def _agg_body(t1, t2, t3, t4, t5, t6, srcm, dstm, zacc,
              acc_out,
              acc_sh, src_v, dst_v, r0, r1, r2, s0, s1, s2):
    c = lax.axis_index("c")
    s = lax.axis_index("s")
    pltpu.sync_copy(zacc.at[pl.ds(s * STRIPE, STRIPE)],
                    acc_sh.at[pl.ds(s * STRIPE, STRIPE)])
    plsc.subcore_barrier()

    base0 = s * KT + jnp.where(c == 0, 0, K0)
    rows = (r0, r1, r2)
    sems = (s0, s1, s2)

    def run(tabs, my_nblk):
        def g(b, j):
            return pltpu.make_async_copy(tabs[b].at[src_v.at[j]], rows[b],
                                         sems[b])

        def outer(bi, carry):
            base = base0 + bi * IBLK
            pltpu.sync_copy(srcm.at[pl.ds(base, IBLK)], src_v)
            pltpu.sync_copy(dstm.at[pl.ds(base, IBLK)], dst_v)
            for j in range(NRING):
                g(j, j).start()
            for j in range(IBLK):
                b = j % NRING
                g(b, j).wait()
                pltpu.sync_copy(rows[b], acc_sh.at[dst_v.at[j]], add=True)
                if j + NRING < IBLK:
                    g(b, j + NRING).start()
            return carry

        lax.fori_loop(0, my_nblk, outer, 0)

    @pl.when(c == 0)
    def _():
        run((t1, t3, t5), K0 // IBLK)

    @pl.when(c != 0)
    def _():
        run((t2, t4, t6), K1 // IBLK)

    plsc.subcore_barrier()
    pltpu.sync_copy(acc_sh.at[pl.ds(s * STRIPE, STRIPE)],
                    acc_out.at[c, pl.ds(s * STRIPE, STRIPE)])


def _cnt_body(dstm, zcnt, ones_hbm,
              cnt_out,
              cnt_sh, dst_v, ones_v):
    c = lax.axis_index("c")
    s = lax.axis_index("s")
    w = s * NC + c
    pltpu.sync_copy(zcnt.at[pl.ds(s * STRIPE, STRIPE)],
                    cnt_sh.at[pl.ds(s * STRIPE, STRIPE)])
    pltpu.sync_copy(ones_hbm, ones_v)
    plsc.subcore_barrier()

    def outer(bi, carry):
        base = w * CPWC + bi * IBLK
        pltpu.sync_copy(dstm.at[pl.ds(base, IBLK)], dst_v)

        def inner(j, c2):
            pltpu.sync_copy(ones_v, cnt_sh.at[dst_v.at[j]], add=True)
            return c2

        lax.fori_loop(0, IBLK, inner, 0)
        return carry

    lax.fori_loop(0, CPWC // IBLK, outer, 0)
    plsc.subcore_barrier()
    pltpu.sync_copy(cnt_sh.at[pl.ds(s * STRIPE, STRIPE)],
                    cnt_out.at[c, pl.ds(s * STRIPE, STRIPE)])


_sc_agg = pl.kernel(
    _agg_body,
    out_type=jax.ShapeDtypeStruct((NC, N_ACC, CC), jnp.float32),
    mesh=plsc.VectorSubcoreMesh(**_MESH),
    scratch_types=[
        pltpu.VMEM_SHARED((N_ACC, CC), jnp.float32),
        pltpu.VMEM((IBLK, CHUNK), jnp.int32),
        pltpu.VMEM((IBLK, CHUNK), jnp.int32),
        pltpu.VMEM((CHUNK, CC), jnp.float32),
        pltpu.VMEM((CHUNK, CC), jnp.float32),
        pltpu.VMEM((CHUNK, CC), jnp.float32),
        pltpu.SemaphoreType.DMA,
        pltpu.SemaphoreType.DMA,
        pltpu.SemaphoreType.DMA,
    ],
)

_sc_cnt = pl.kernel(
    _cnt_body,
    out_type=jax.ShapeDtypeStruct((NC, N_ACC, CC), jnp.float32),
    mesh=plsc.VectorSubcoreMesh(**_MESH),
    scratch_types=[
        pltpu.VMEM_SHARED((N_ACC, CC), jnp.float32),
        pltpu.VMEM((IBLK, CHUNKC), jnp.int32),
        pltpu.VMEM((CHUNKC, CC), jnp.float32),
    ],
)


def _tc_layer_body(relu, acc_ref, cnt_ref, x_ref, wl_ref, wr_ref, b_ref, o_ref):
    acc = acc_ref[0] + acc_ref[1]
    cnt = cnt_ref[0] + cnt_ref[1]
    denom = jnp.maximum(cnt[:, 0:1], 1.0)
    agg = acc / denom
    h = (jnp.dot(agg, wl_ref[...], preferred_element_type=jnp.float32)
         + jnp.dot(x_ref[...], wr_ref[...], preferred_element_type=jnp.float32)
         + b_ref[...])
    o_ref[...] = jnp.maximum(h, 0.0) if relu else h


def _tc_layer(relu, acc, cnt, x, wl, wr, b):
    blk = 1000
    grid = (NN // blk,)
    return pl.pallas_call(
        functools.partial(_tc_layer_body, relu),
        grid=grid,
        in_specs=[
            pl.BlockSpec((NC, blk, CC), lambda i: (0, i, 0)),
            pl.BlockSpec((NC, blk, CC), lambda i: (0, i, 0)),
            pl.BlockSpec((blk, CC), lambda i: (i, 0)),
            pl.BlockSpec((CC, CC), lambda i: (0, 0)),
            pl.BlockSpec((CC, CC), lambda i: (0, 0)),
            pl.BlockSpec((1, CC), lambda i: (0, 0)),
        ],
        out_specs=pl.BlockSpec((blk, CC), lambda i: (i, 0)),
        out_shape=jax.ShapeDtypeStruct((NN, CC), jnp.float32),
    )(acc, cnt, x, wl, wr, b)


def kernel(x, edge_index, W1_l, W1_r, b1, W2_l, W2_r, b2):
    src = edge_index[0]
    dst = edge_index[1]
    pad = E_PAD - EE
    srcm = jnp.concatenate([src, jnp.zeros((pad,), jnp.int32)]).reshape(
        IDX_ROWS, CHUNK)
    dst_pad = jnp.concatenate([dst, jnp.full((pad,), NN, jnp.int32)])
    dstm = dst_pad.reshape(IDX_ROWS, CHUNK)
    dstmc = dst_pad.reshape(IDX_ROWS_C, CHUNKC)
    zacc = jnp.zeros((N_ACC, CC), jnp.float32)
    ones = jnp.ones((CHUNKC, CC), jnp.float32)

    zs = lax.optimization_barrier(tuple(jnp.float32(0.0) for _ in range(5)))
    xs = tuple(x + z for z in zs)
    cnt = _sc_cnt(dstmc, zacc, ones)
    acc1 = _sc_agg(x, *xs, srcm, dstm, zacc)
    h = _tc_layer(True, acc1, cnt, x, W1_l, W1_r, b1.reshape(1, CC))
    hs = tuple(h + z for z in zs)
    acc2 = _sc_agg(h, *hs, srcm, dstm, zacc)
    out = _tc_layer(False, acc2, cnt, h, W2_l, W2_r, b2.reshape(1, CC))
    return out
